# baseline (speedup 1.0000x reference)
import functools

import jax
import jax.numpy as jnp
from jax import lax
from jax.experimental import pallas as pl
from jax.experimental.pallas import tpu as pltpu
from jax.experimental.pallas import tpu_sc as plsc

N = 10000
E = 160000
D_IN = 256
D_H = 256
N_CLS = 16
P = 8192

HALF = 128
NT = 16
CHUNK = 80
NCHUNK = (E // NT) // CHUNK
NG = 5
GC = NCHUNK // NG
RING = 4
RCHUNK = N // CHUNK

_MESH = dict(core_axis_name="c", subcore_axis_name="s")


def _make_agg(mul: int, coff: int, with_deg: bool):
  out_type = [jax.ShapeDtypeStruct((2, N, HALF), jnp.float32)]
  if with_deg:
    out_type.append(jax.ShapeDtypeStruct((N,), jnp.float32))

  scratch = [
      pltpu.VMEM((GC, CHUNK), jnp.int32),
      pltpu.VMEM((GC, CHUNK), jnp.int32),
      pltpu.VMEM((RING, CHUNK, HALF), jnp.float32),
      pltpu.VMEM((CHUNK,), jnp.float32),
      pltpu.VMEM_SHARED((N, HALF), jnp.float32),
      pltpu.VMEM_SHARED((N,), jnp.float32),
  ] + [pltpu.SemaphoreType.DMA] * (2 * RING)

  def body(table, src_h, dst_h, *rest):
    if with_deg:
      out, deg_out = rest[0], rest[1]
      rest = rest[2:]
    else:
      out = rest[0]
      rest = rest[1:]
    idx_s, idx_d, rows, vec80, acc, dacc = rest[:6]
    gsem = rest[6:6 + RING]
    ssem = rest[6 + RING:]

    c = lax.axis_index("c")
    s = lax.axis_index("s")

    off = (c * coff).astype(jnp.int32)

    def zrow(j, _):
      for k in range(HALF // 16):
        rows[0, j, pl.ds(k * 16, 16)] = jnp.zeros((16,), jnp.float32)
      return _
    lax.fori_loop(0, CHUNK, zrow, None)
    lo = s * 8
    hi = jnp.minimum((s + 1) * 8, RCHUNK)

    def zacc(j, _):
      pltpu.sync_copy(rows.at[0], acc.at[pl.ds(j * CHUNK, CHUNK)])
      return _
    lax.fori_loop(lo, hi, zacc, None)

    if with_deg:
      for k in range(CHUNK // 16):
        vec80[pl.ds(k * 16, 16)] = jnp.zeros((16,), jnp.float32)

      @pl.when(c == 0)
      def _zero_deg():
        def zd(j, _):
          pltpu.sync_copy(vec80, dacc.at[pl.ds(j * CHUNK, CHUNK)])
          return _
        lax.fori_loop(s * 8, jnp.minimum((s + 1) * 8, NCHUNK), zd, None)

    plsc.subcore_barrier()

    if with_deg:
      for k in range(CHUNK // 16):
        vec80[pl.ds(k * 16, 16)] = jnp.ones((16,), jnp.float32)

    def wait_g(j, b):
      pltpu.make_async_copy(table.at[idx_s.at[j]], rows.at[b],
                            gsem[b]).wait()

    def s_issue(j, b):
      pltpu.async_copy(rows.at[b], acc.at[idx_d.at[j]], ssem[b], add=True)
      if with_deg:
        @pl.when(c == 0)
        def _deg():
          pltpu.sync_copy(vec80, dacc.at[idx_d.at[j]], add=True)

    def wait_s(b):
      pltpu.make_async_copy(rows.at[b], acc.at[idx_d.at[0]],
                            ssem[b]).wait()

    def g_issue(j, b):
      pltpu.async_copy(table.at[idx_s.at[j]], rows.at[b], gsem[b])

    def group(g, _):
      pltpu.sync_copy(src_h.at[s, g], idx_s)
      pltpu.sync_copy(dst_h.at[s, g], idx_d)

      def adj(j, _):
        for k in range(CHUNK // 16):
          sl = pl.ds(k * 16, 16)
          idx_s[j, sl] = idx_s[j, sl] * mul + off
        return _
      lax.fori_loop(0, GC, adj, None)

      g_issue(0, 0)
      g_issue(1, 1)
      wait_g(0, 0); s_issue(0, 0); g_issue(2, 2)
      wait_g(1, 1); s_issue(1, 1); g_issue(3, 3)

      @pl.loop(2, GC - 3, step=RING)
      def step(i):
        for k in range(RING):
          j = i + k
          b = (2 + k) % RING
          wait_g(j, b)
          s_issue(j, b)
          wait_s((b + 2) % RING)
          g_issue(j + 2, (b + 2) % RING)

      wait_g(GC - 3, 2); s_issue(GC - 3, 2); wait_s(0); g_issue(GC - 1, 0)
      wait_g(GC - 2, 3); s_issue(GC - 2, 3)
      wait_g(GC - 1, 0); s_issue(GC - 1, 0)
      for b in (1, 2, 3, 0):
        wait_s(b)
      return _

    lax.fori_loop(0, NG, group, None)

    plsc.subcore_barrier()

    def cout(j, _):
      pltpu.sync_copy(acc.at[pl.ds(j * CHUNK, CHUNK)], rows.at[0])
      pltpu.sync_copy(rows.at[0], out.at[c, pl.ds(j * CHUNK, CHUNK)])
      return _
    lax.fori_loop(lo, hi, cout, None)

    if with_deg:
      @pl.when(c == 0)
      def _deg_out():
        def dout(j, _):
          pltpu.sync_copy(dacc.at[pl.ds(j * CHUNK, CHUNK)], vec80)
          pltpu.sync_copy(vec80, deg_out.at[pl.ds(j * CHUNK, CHUNK)])
          return _
        lax.fori_loop(s * 8, jnp.minimum((s + 1) * 8, NCHUNK), dout, None)

  return pl.kernel(
      body,
      out_type=out_type,
      mesh=plsc.VectorSubcoreMesh(**_MESH),
      scratch_types=scratch,
      name=f"sage_agg_deg{int(with_deg)}",
  )


def _pair_gather_kernel(z, idxp):
  scratch = [
      pltpu.VMEM((P // NT // CHUNK_PG, CHUNK_PG), jnp.int32),
      pltpu.VMEM((CHUNK_PG, HALF), jnp.float32),
      pltpu.SemaphoreType.DMA,
  ]

  def body(z_h, idx_h, out, idxv, rows, sem):
    c = lax.axis_index("c")
    s = lax.axis_index("s")
    pltpu.sync_copy(idx_h.at[c, s], idxv)

    def step(t, _):
      pltpu.async_copy(z_h.at[idxv.at[t]], rows, sem).wait()
      pltpu.sync_copy(rows, out.at[c, pl.ds(s * (P // NT) + t * CHUNK_PG,
                                            CHUNK_PG)])
      return _
    lax.fori_loop(0, P // NT // CHUNK_PG, step, None)

  return pl.kernel(
      body,
      out_type=jax.ShapeDtypeStruct((2, P, HALF), jnp.float32),
      mesh=plsc.VectorSubcoreMesh(**_MESH),
      scratch_types=scratch,
      name="pair_gather",
  )(z, idxp)


CHUNK_PG = 128


def _self_body(u, wt, b, out):
  out[...] = (jnp.dot(u[...], wt[...], preferred_element_type=jnp.float32)
              + b[...])


def _self2_body(h1, wt, b, out):
  hp = jnp.concatenate([h1[0], h1[1]], axis=1)
  out[...] = (jnp.dot(hp, wt[...], preferred_element_type=jnp.float32)
              + b[...])


def _tc1_body(xr, agg, deg, wlt, out):
  a = jnp.concatenate([agg[0], agg[1]], axis=1)
  a = a / jnp.maximum(deg[...], 1.0)
  h = jnp.dot(a, wlt[...], preferred_element_type=jnp.float32) + xr[...]
  col_mean = jnp.mean(h, axis=0, keepdims=True)
  rn = jnp.sqrt(1e-6 + jnp.sum(h * h, axis=1, keepdims=True))
  h = jnp.maximum(h / rn - col_mean, 0.0)
  out[0] = h[:, :HALF]
  out[1] = h[:, HALF:]


def _tc2_body(agg, deg, hr, wlt, clf_wt, clf_b,
              bn1_g, bn1_b, ls1_wt, bn2_g, bn2_b, ls2_wt,
              logp_out, z_out):
  a = jnp.concatenate([agg[0], agg[1]], axis=1)
  a = a / jnp.maximum(deg[...], 1.0)
  h = jnp.dot(a, wlt[...], preferred_element_type=jnp.float32) + hr[...]

  logits = (jnp.dot(jnp.maximum(h, 0.0), clf_wt[...],
                    preferred_element_type=jnp.float32) + clf_b[...])
  m = jnp.max(logits, axis=1, keepdims=True)
  lse = jnp.log(jnp.sum(jnp.exp(logits - m), axis=1, keepdims=True)) + m
  logp_out[...] = logits - lse

  rs = 1.0 / jnp.sqrt(1.0 + 1e-5)
  z = h * (bn1_g[...] * rs) + bn1_b[...]
  z = jnp.dot(z, ls1_wt[...], preferred_element_type=jnp.float32)
  z = jnp.tanh(z * (bn2_g[...] * rs) + bn2_b[...])
  z_out[...] = jnp.dot(z, ls2_wt[...], preferred_element_type=jnp.float32)


def _tc3_body(zz, ba1_wt, ba1_b, ba2_wt, ba2_b, out):
  za, zb = zz[0], zz[1]

  def biasatt(u):
    t = jnp.tanh(jnp.dot(u, ba1_wt[...],
                         preferred_element_type=jnp.float32) + ba1_b[...])
    return jnp.dot(t, ba2_wt[...],
                   preferred_element_type=jnp.float32) + ba2_b[...]

  a = za + biasatt(za)
  b = zb + biasatt(zb)
  na = jnp.maximum(jnp.sqrt(jnp.sum(a * a, axis=1)), 1e-8)
  nb = jnp.maximum(jnp.sqrt(jnp.sum(b * b, axis=1)), 1e-8)
  alpha = jax.nn.sigmoid(jnp.sum(a * b, axis=1) / (na * nb))
  out[...] = alpha.reshape(P, 1)


@jax.jit
def kernel(x, edge_index, idx1, idx2, params):
  p = params
  ei = edge_index.astype(jnp.int32)
  src = ei[0].reshape(NT, NG, GC, CHUNK)
  dst = ei[1].reshape(NT, NG, GC, CHUNK)

  agg1, deg = _make_agg(2, 1, True)(x.reshape(2 * N, HALF), src, dst)
  xr = pl.pallas_call(
      _self_body,
      out_shape=jax.ShapeDtypeStruct((N, D_H), jnp.float32),
  )(x, p['c1_Wr'].T, p['c1_bl'])
  deg2 = deg.reshape(N, 1)

  h1 = pl.pallas_call(
      _tc1_body,
      out_shape=jax.ShapeDtypeStruct((2, N, HALF), jnp.float32),
  )(xr, agg1, deg2, p['c1_Wl'].T)

  agg2 = _make_agg(1, N, False)(h1.reshape(2 * N, HALF), src, dst)[0]
  hr = pl.pallas_call(
      _self2_body,
      out_shape=jax.ShapeDtypeStruct((N, D_H), jnp.float32),
  )(h1, p['c2_Wr'].T, p['c2_bl'])

  logp, z = pl.pallas_call(
      _tc2_body,
      out_shape=[jax.ShapeDtypeStruct((N, N_CLS), jnp.float32),
                 jax.ShapeDtypeStruct((N, HALF), jnp.float32)],
  )(agg2, deg2, hr, p['c2_Wl'].T,
    p['clf_W'].T, p['clf_b'], p['bn1_g'], p['bn1_b'], p['ls1_W'].T,
    p['bn2_g'], p['bn2_b'], p['ls2_W'].T)

  idxp = jnp.stack([idx1, idx2]).astype(jnp.int32).reshape(
      2, NT, P // NT // CHUNK_PG, CHUNK_PG)
  zz = _pair_gather_kernel(z, idxp)

  alpha = pl.pallas_call(
      _tc3_body,
      out_shape=jax.ShapeDtypeStruct((P, 1), jnp.float32),
  )(zz, p['ba1_W'].T, p['ba1_b'], p['ba2_W'].T, p['ba2_b'])

  return (alpha, logp)


# speedup vs baseline: 1.0795x; 1.0795x over previous
import functools

import jax
import jax.numpy as jnp
from jax import lax
from jax.experimental import pallas as pl
from jax.experimental.pallas import tpu as pltpu
from jax.experimental.pallas import tpu_sc as plsc

N = 10000
E = 160000
D_IN = 256
D_H = 256
N_CLS = 16
P = 8192

HALF = 128
NT = 16
CHUNK = 80
NCHUNK = (E // NT) // CHUNK
NG = 5
GC = NCHUNK // NG
RING = 4
RCHUNK = N // CHUNK

_MESH = dict(core_axis_name="c", subcore_axis_name="s")


def _make_agg(with_deg: bool):
  out_type = [jax.ShapeDtypeStruct((2, N, HALF), jnp.float32)]
  if with_deg:
    out_type.append(jax.ShapeDtypeStruct((N,), jnp.float32))

  scratch = [
      pltpu.VMEM((GC, CHUNK), jnp.int32),
      pltpu.VMEM((GC, CHUNK), jnp.int32),
      pltpu.VMEM((RING, CHUNK, HALF), jnp.float32),
      pltpu.VMEM((CHUNK,), jnp.float32),
      pltpu.VMEM_SHARED((N, HALF), jnp.float32),
      pltpu.VMEM_SHARED((N,), jnp.float32),
  ] + [pltpu.SemaphoreType.DMA] * RING

  def body(table, src_h, dst_h, *rest):
    if with_deg:
      out, deg_out = rest[0], rest[1]
      rest = rest[2:]
    else:
      out = rest[0]
      rest = rest[1:]
    idx_s, idx_d, rows, vec80, acc, dacc = rest[:6]
    sems = rest[6:]

    c = lax.axis_index("c")
    s = lax.axis_index("s")

    plane = table.at[c]

    def zrow(j, _):
      for k in range(HALF // 16):
        rows[0, j, pl.ds(k * 16, 16)] = jnp.zeros((16,), jnp.float32)
      return _
    lax.fori_loop(0, CHUNK, zrow, None)
    lo = s * 8
    hi = jnp.minimum((s + 1) * 8, RCHUNK)

    def zacc(j, _):
      pltpu.sync_copy(rows.at[0], acc.at[pl.ds(j * CHUNK, CHUNK)])
      return _
    lax.fori_loop(lo, hi, zacc, None)

    if with_deg:
      for k in range(CHUNK // 16):
        vec80[pl.ds(k * 16, 16)] = jnp.zeros((16,), jnp.float32)

      @pl.when(c == 0)
      def _zero_deg():
        def zd(j, _):
          pltpu.sync_copy(vec80, dacc.at[pl.ds(j * CHUNK, CHUNK)])
          return _
        lax.fori_loop(s * 8, jnp.minimum((s + 1) * 8, NCHUNK), zd, None)

    plsc.subcore_barrier()

    if with_deg:
      for k in range(CHUNK // 16):
        vec80[pl.ds(k * 16, 16)] = jnp.ones((16,), jnp.float32)

    def consume(j, b):
      pltpu.make_async_copy(plane.at[idx_s.at[j]], rows.at[b],
                            sems[b]).wait()
      pltpu.sync_copy(rows.at[b], acc.at[idx_d.at[j]], add=True)
      if with_deg:
        @pl.when(c == 0)
        def _deg():
          pltpu.sync_copy(vec80, dacc.at[idx_d.at[j]], add=True)

    def group(g, _):
      pltpu.sync_copy(src_h.at[s, g], idx_s)
      pltpu.sync_copy(dst_h.at[s, g], idx_d)

      for b in range(RING):
        pltpu.async_copy(plane.at[idx_s.at[b]], rows.at[b], sems[b])

      @pl.loop(0, GC - 2 * RING + 1, step=RING)
      def step(i):
        for b in range(RING):
          consume(i + b, b)
          pltpu.async_copy(plane.at[idx_s.at[i + b + RING]], rows.at[b],
                           sems[b])

      jt = GC - RING - 1
      consume(jt, jt % RING)
      pltpu.async_copy(plane.at[idx_s.at[GC - 1]], rows.at[(GC - 1) % RING],
                       sems[(GC - 1) % RING])
      for j in range(GC - RING, GC):
        consume(j, j % RING)
      return _

    lax.fori_loop(0, NG, group, None)

    plsc.subcore_barrier()

    def cout(j, _):
      pltpu.sync_copy(acc.at[pl.ds(j * CHUNK, CHUNK)],
                      out.at[c, pl.ds(j * CHUNK, CHUNK)])
      return _
    lax.fori_loop(lo, hi, cout, None)

    if with_deg:
      @pl.when(c == 0)
      def _deg_out():
        def dout(j, _):
          pltpu.sync_copy(dacc.at[pl.ds(j * CHUNK, CHUNK)], vec80)
          pltpu.sync_copy(vec80, deg_out.at[pl.ds(j * CHUNK, CHUNK)])
          return _
        lax.fori_loop(s * 8, jnp.minimum((s + 1) * 8, NCHUNK), dout, None)

  return pl.kernel(
      body,
      out_type=out_type,
      mesh=plsc.VectorSubcoreMesh(**_MESH),
      scratch_types=scratch,
      name=f"sage_agg_deg{int(with_deg)}",
  )


def _pair_gather_kernel(z, idxp):
  scratch = [
      pltpu.VMEM((P // NT // CHUNK_PG, CHUNK_PG), jnp.int32),
      pltpu.VMEM((CHUNK_PG, HALF), jnp.float32),
      pltpu.SemaphoreType.DMA,
  ]

  def body(z_h, idx_h, out, idxv, rows, sem):
    c = lax.axis_index("c")
    s = lax.axis_index("s")
    pltpu.sync_copy(idx_h.at[c, s], idxv)

    def step(t, _):
      pltpu.async_copy(z_h.at[idxv.at[t]], rows, sem).wait()
      pltpu.sync_copy(rows, out.at[c, pl.ds(s * (P // NT) + t * CHUNK_PG,
                                            CHUNK_PG)])
      return _
    lax.fori_loop(0, P // NT // CHUNK_PG, step, None)

  return pl.kernel(
      body,
      out_type=jax.ShapeDtypeStruct((2, P, HALF), jnp.float32),
      mesh=plsc.VectorSubcoreMesh(**_MESH),
      scratch_types=scratch,
      name="pair_gather",
  )(z, idxp)


CHUNK_PG = 128


def _tc1_body(x, agg, deg, wlt, bl, wrt, out):
  a = jnp.concatenate([agg[0], agg[1]], axis=1)
  a = a / jnp.maximum(deg[...], 1.0)
  h = (jnp.dot(a, wlt[...], preferred_element_type=jnp.float32) + bl[...]
       + jnp.dot(x[...], wrt[...], preferred_element_type=jnp.float32))
  col_mean = jnp.mean(h, axis=0, keepdims=True)
  rn = jnp.sqrt(1e-6 + jnp.sum(h * h, axis=1, keepdims=True))
  h = jnp.maximum(h / rn - col_mean, 0.0)
  out[0] = h[:, :HALF]
  out[1] = h[:, HALF:]


def _tc2_body(agg, deg, h1, wlt, bl, wrt, clf_wt, clf_b,
              bn1_g, bn1_b, ls1_wt, bn2_g, bn2_b, ls2_wt,
              logp_out, z_out):
  a = jnp.concatenate([agg[0], agg[1]], axis=1)
  a = a / jnp.maximum(deg[...], 1.0)
  hp = jnp.concatenate([h1[0], h1[1]], axis=1)
  h = (jnp.dot(a, wlt[...], preferred_element_type=jnp.float32) + bl[...]
       + jnp.dot(hp, wrt[...], preferred_element_type=jnp.float32))

  logits = (jnp.dot(jnp.maximum(h, 0.0), clf_wt[...],
                    preferred_element_type=jnp.float32) + clf_b[...])
  m = jnp.max(logits, axis=1, keepdims=True)
  lse = jnp.log(jnp.sum(jnp.exp(logits - m), axis=1, keepdims=True)) + m
  logp_out[...] = logits - lse

  rs = 1.0 / jnp.sqrt(1.0 + 1e-5)
  z = h * (bn1_g[...] * rs) + bn1_b[...]
  z = jnp.dot(z, ls1_wt[...], preferred_element_type=jnp.float32)
  z = jnp.tanh(z * (bn2_g[...] * rs) + bn2_b[...])
  z_out[...] = jnp.dot(z, ls2_wt[...], preferred_element_type=jnp.float32)


def _tc3_body(zz, ba1_wt, ba1_b, ba2_wt, ba2_b, out):
  za, zb = zz[0], zz[1]

  def biasatt(u):
    t = jnp.tanh(jnp.dot(u, ba1_wt[...],
                         preferred_element_type=jnp.float32) + ba1_b[...])
    return jnp.dot(t, ba2_wt[...],
                   preferred_element_type=jnp.float32) + ba2_b[...]

  a = za + biasatt(za)
  b = zb + biasatt(zb)
  na = jnp.maximum(jnp.sqrt(jnp.sum(a * a, axis=1)), 1e-8)
  nb = jnp.maximum(jnp.sqrt(jnp.sum(b * b, axis=1)), 1e-8)
  alpha = jax.nn.sigmoid(jnp.sum(a * b, axis=1) / (na * nb))
  out[...] = alpha.reshape(P, 1)


@jax.jit
def kernel(x, edge_index, idx1, idx2, params):
  p = params
  ei = edge_index.astype(jnp.int32)
  src = ei[0].reshape(NT, NG, GC, CHUNK)
  dst = ei[1].reshape(NT, NG, GC, CHUNK)

  xt = x.reshape(N, 2, HALF).transpose(1, 0, 2)
  agg1, deg = _make_agg(True)(xt, src, dst)
  deg2 = deg.reshape(N, 1)

  h1 = pl.pallas_call(
      _tc1_body,
      out_shape=jax.ShapeDtypeStruct((2, N, HALF), jnp.float32),
  )(x, agg1, deg2, p['c1_Wl'].T, p['c1_bl'], p['c1_Wr'].T)

  agg2 = _make_agg(False)(h1, src, dst)[0]

  logp, z = pl.pallas_call(
      _tc2_body,
      out_shape=[jax.ShapeDtypeStruct((N, N_CLS), jnp.float32),
                 jax.ShapeDtypeStruct((N, HALF), jnp.float32)],
  )(agg2, deg2, h1, p['c2_Wl'].T, p['c2_bl'], p['c2_Wr'].T,
    p['clf_W'].T, p['clf_b'], p['bn1_g'], p['bn1_b'], p['ls1_W'].T,
    p['bn2_g'], p['bn2_b'], p['ls2_W'].T)

  idxp = jnp.stack([idx1, idx2]).astype(jnp.int32).reshape(
      2, NT, P // NT // CHUNK_PG, CHUNK_PG)
  zz = _pair_gather_kernel(z, idxp)

  alpha = pl.pallas_call(
      _tc3_body,
      out_shape=jax.ShapeDtypeStruct((P, 1), jnp.float32),
  )(zz, p['ba1_W'].T, p['ba1_b'], p['ba2_W'].T, p['ba2_b'])

  return (alpha, logp)


# speedup vs baseline: 1.0970x; 1.0162x over previous
import functools

import jax
import jax.numpy as jnp
from jax import lax
from jax.experimental import pallas as pl
from jax.experimental.pallas import tpu as pltpu
from jax.experimental.pallas import tpu_sc as plsc

N = 10000
E = 160000
D_IN = 256
D_H = 256
N_CLS = 16
P = 8192

HALF = 128
NT = 16
CHUNK = 80
NCHUNK = (E // NT) // CHUNK
NG = 5
GC = NCHUNK // NG
RING = 4
RCHUNK = N // CHUNK

_MESH = dict(core_axis_name="c", subcore_axis_name="s")


def _make_agg(mul: int, with_deg: bool):
  out_type = [jax.ShapeDtypeStruct((2, N, HALF), jnp.float32)]
  if with_deg:
    out_type.append(jax.ShapeDtypeStruct((N,), jnp.float32))

  scratch = [
      pltpu.VMEM((GC, CHUNK), jnp.int32),
      pltpu.VMEM((GC, CHUNK), jnp.int32),
      pltpu.VMEM((RING, CHUNK, HALF), jnp.float32),
      pltpu.VMEM((CHUNK,), jnp.float32),
      pltpu.VMEM_SHARED((N, HALF), jnp.float32),
      pltpu.VMEM_SHARED((N,), jnp.float32),
  ] + [pltpu.SemaphoreType.DMA] * RING

  def body(table, src_h, dst_h, *rest):
    if with_deg:
      out, deg_out = rest[0], rest[1]
      rest = rest[2:]
    else:
      out = rest[0]
      rest = rest[1:]
    idx_s, idx_d, rows, vec80, acc, dacc = rest[:6]
    sems = rest[6:]

    c = lax.axis_index("c")
    s = lax.axis_index("s")

    plane = table if mul else table.at[c]
    off = (c * 1).astype(jnp.int32) if mul else None

    def zrow(j, _):
      for k in range(HALF // 16):
        rows[0, j, pl.ds(k * 16, 16)] = jnp.zeros((16,), jnp.float32)
      return _
    lax.fori_loop(0, CHUNK, zrow, None)
    lo = s * 8
    hi = jnp.minimum((s + 1) * 8, RCHUNK)

    def zacc(j, _):
      pltpu.sync_copy(rows.at[0], acc.at[pl.ds(j * CHUNK, CHUNK)])
      return _
    lax.fori_loop(lo, hi, zacc, None)

    if with_deg:
      for k in range(CHUNK // 16):
        vec80[pl.ds(k * 16, 16)] = jnp.zeros((16,), jnp.float32)

      @pl.when(c == 0)
      def _zero_deg():
        def zd(j, _):
          pltpu.sync_copy(vec80, dacc.at[pl.ds(j * CHUNK, CHUNK)])
          return _
        lax.fori_loop(s * 8, jnp.minimum((s + 1) * 8, NCHUNK), zd, None)

    plsc.subcore_barrier()

    if with_deg:
      for k in range(CHUNK // 16):
        vec80[pl.ds(k * 16, 16)] = jnp.ones((16,), jnp.float32)

    def consume(j, b):
      pltpu.make_async_copy(plane.at[idx_s.at[j]], rows.at[b],
                            sems[b]).wait()
      pltpu.sync_copy(rows.at[b], acc.at[idx_d.at[j]], add=True)
      if with_deg:
        @pl.when(c == 0)
        def _deg():
          pltpu.sync_copy(vec80, dacc.at[idx_d.at[j]], add=True)

    def group(g, _):
      pltpu.sync_copy(src_h.at[s, g], idx_s)
      pltpu.sync_copy(dst_h.at[s, g], idx_d)
      if mul:
        def adj(j, _):
          for k in range(CHUNK // 16):
            sl = pl.ds(k * 16, 16)
            idx_s[j, sl] = idx_s[j, sl] * mul + off
          return _
        lax.fori_loop(0, GC, adj, None)

      for b in range(RING):
        pltpu.async_copy(plane.at[idx_s.at[b]], rows.at[b], sems[b])

      @pl.loop(0, GC - 2 * RING + 1, step=RING)
      def step(i):
        for b in range(RING):
          consume(i + b, b)
          pltpu.async_copy(plane.at[idx_s.at[i + b + RING]], rows.at[b],
                           sems[b])

      jt = GC - RING - 1
      consume(jt, jt % RING)
      pltpu.async_copy(plane.at[idx_s.at[GC - 1]], rows.at[(GC - 1) % RING],
                       sems[(GC - 1) % RING])
      for j in range(GC - RING, GC):
        consume(j, j % RING)
      return _

    lax.fori_loop(0, NG, group, None)

    plsc.subcore_barrier()

    def cout(j, _):
      pltpu.sync_copy(acc.at[pl.ds(j * CHUNK, CHUNK)],
                      out.at[c, pl.ds(j * CHUNK, CHUNK)])
      return _
    lax.fori_loop(lo, hi, cout, None)

    if with_deg:
      @pl.when(c == 0)
      def _deg_out():
        def dout(j, _):
          pltpu.sync_copy(dacc.at[pl.ds(j * CHUNK, CHUNK)], vec80)
          pltpu.sync_copy(vec80, deg_out.at[pl.ds(j * CHUNK, CHUNK)])
          return _
        lax.fori_loop(s * 8, jnp.minimum((s + 1) * 8, NCHUNK), dout, None)

  return pl.kernel(
      body,
      out_type=out_type,
      mesh=plsc.VectorSubcoreMesh(**_MESH),
      scratch_types=scratch,
      name=f"sage_agg_deg{int(with_deg)}",
  )


def _pair_gather_kernel(z, idxp):
  scratch = [
      pltpu.VMEM((P // NT // CHUNK_PG, CHUNK_PG), jnp.int32),
      pltpu.VMEM((CHUNK_PG, HALF), jnp.float32),
      pltpu.SemaphoreType.DMA,
  ]

  def body(z_h, idx_h, out, idxv, rows, sem):
    c = lax.axis_index("c")
    s = lax.axis_index("s")
    pltpu.sync_copy(idx_h.at[c, s], idxv)

    def step(t, _):
      pltpu.async_copy(z_h.at[idxv.at[t]], rows, sem).wait()
      pltpu.sync_copy(rows, out.at[c, pl.ds(s * (P // NT) + t * CHUNK_PG,
                                            CHUNK_PG)])
      return _
    lax.fori_loop(0, P // NT // CHUNK_PG, step, None)

  return pl.kernel(
      body,
      out_type=jax.ShapeDtypeStruct((2, P, HALF), jnp.float32),
      mesh=plsc.VectorSubcoreMesh(**_MESH),
      scratch_types=scratch,
      name="pair_gather",
  )(z, idxp)


CHUNK_PG = 128


def _tc1_body(x, agg, deg, wlt, bl, wrt, out):
  a = jnp.concatenate([agg[0], agg[1]], axis=1)
  a = a / jnp.maximum(deg[...], 1.0)
  h = (jnp.dot(a, wlt[...], preferred_element_type=jnp.float32) + bl[...]
       + jnp.dot(x[...], wrt[...], preferred_element_type=jnp.float32))
  col_mean = jnp.mean(h, axis=0, keepdims=True)
  rn = jnp.sqrt(1e-6 + jnp.sum(h * h, axis=1, keepdims=True))
  h = jnp.maximum(h / rn - col_mean, 0.0)
  out[0] = h[:, :HALF]
  out[1] = h[:, HALF:]


def _tc2_body(agg, deg, h1, wlt, bl, wrt, clf_wt, clf_b,
              bn1_g, bn1_b, ls1_wt, bn2_g, bn2_b, ls2_wt,
              logp_out, z_out):
  a = jnp.concatenate([agg[0], agg[1]], axis=1)
  a = a / jnp.maximum(deg[...], 1.0)
  hp = jnp.concatenate([h1[0], h1[1]], axis=1)
  h = (jnp.dot(a, wlt[...], preferred_element_type=jnp.float32) + bl[...]
       + jnp.dot(hp, wrt[...], preferred_element_type=jnp.float32))

  logits = (jnp.dot(jnp.maximum(h, 0.0), clf_wt[...],
                    preferred_element_type=jnp.float32) + clf_b[...])
  m = jnp.max(logits, axis=1, keepdims=True)
  lse = jnp.log(jnp.sum(jnp.exp(logits - m), axis=1, keepdims=True)) + m
  logp_out[...] = logits - lse

  rs = 1.0 / jnp.sqrt(1.0 + 1e-5)
  z = h * (bn1_g[...] * rs) + bn1_b[...]
  z = jnp.dot(z, ls1_wt[...], preferred_element_type=jnp.float32)
  z = jnp.tanh(z * (bn2_g[...] * rs) + bn2_b[...])
  z_out[...] = jnp.dot(z, ls2_wt[...], preferred_element_type=jnp.float32)


def _tc3_body(zz, ba1_wt, ba1_b, ba2_wt, ba2_b, out):
  za, zb = zz[0], zz[1]

  def biasatt(u):
    t = jnp.tanh(jnp.dot(u, ba1_wt[...],
                         preferred_element_type=jnp.float32) + ba1_b[...])
    return jnp.dot(t, ba2_wt[...],
                   preferred_element_type=jnp.float32) + ba2_b[...]

  a = za + biasatt(za)
  b = zb + biasatt(zb)
  na = jnp.maximum(jnp.sqrt(jnp.sum(a * a, axis=1)), 1e-8)
  nb = jnp.maximum(jnp.sqrt(jnp.sum(b * b, axis=1)), 1e-8)
  alpha = jax.nn.sigmoid(jnp.sum(a * b, axis=1) / (na * nb))
  out[...] = alpha.reshape(P, 1)


@jax.jit
def kernel(x, edge_index, idx1, idx2, params):
  p = params
  ei = edge_index.astype(jnp.int32)
  src = ei[0].reshape(NT, NG, GC, CHUNK)
  dst = ei[1].reshape(NT, NG, GC, CHUNK)

  agg1, deg = _make_agg(2, True)(x.reshape(2 * N, HALF), src, dst)
  deg2 = deg.reshape(N, 1)

  h1 = pl.pallas_call(
      _tc1_body,
      out_shape=jax.ShapeDtypeStruct((2, N, HALF), jnp.float32),
  )(x, agg1, deg2, p['c1_Wl'].T, p['c1_bl'], p['c1_Wr'].T)

  agg2 = _make_agg(0, False)(h1, src, dst)[0]

  logp, z = pl.pallas_call(
      _tc2_body,
      out_shape=[jax.ShapeDtypeStruct((N, N_CLS), jnp.float32),
                 jax.ShapeDtypeStruct((N, HALF), jnp.float32)],
  )(agg2, deg2, h1, p['c2_Wl'].T, p['c2_bl'], p['c2_Wr'].T,
    p['clf_W'].T, p['clf_b'], p['bn1_g'], p['bn1_b'], p['ls1_W'].T,
    p['bn2_g'], p['bn2_b'], p['ls2_W'].T)

  idxp = jnp.stack([idx1, idx2]).astype(jnp.int32).reshape(
      2, NT, P // NT // CHUNK_PG, CHUNK_PG)
  zz = _pair_gather_kernel(z, idxp)

  alpha = pl.pallas_call(
      _tc3_body,
      out_shape=jax.ShapeDtypeStruct((P, 1), jnp.float32),
  )(zz, p['ba1_W'].T, p['ba1_b'], p['ba2_W'].T, p['ba2_b'])

  return (alpha, logp)
